# baseline (speedup 1.0000x reference)
import jax
import jax.numpy as jnp
from jax import lax
from jax.experimental import pallas as pl
from jax.experimental.pallas import tpu as pltpu
from jax.experimental.pallas import tpu_sc as plsc

_TIME_STEP = 0.002
_SIZE = 100
_NC, _NS = 2, 16
_NW = _NC * _NS
_C, _H, _W = 4, 256, 256
_SPATIAL = _H * _W
_S_PER_W = _C * _SPATIAL // _NW
_CHUNK = 512
_N_CHUNKS = _S_PER_W // _CHUNK
_LANES = 16
_GROUPS = _CHUNK // _LANES


def _sc_body(spikes_hbm, out_hbm, spikes_v, buf):
    wid = lax.axis_index("s") * _NC + lax.axis_index("c")
    base = wid * _S_PER_W
    ch = base // _SPATIAL
    s0 = base % _SPATIAL

    pltpu.sync_copy(spikes_hbm.at[pl.ds(base, _S_PER_W)], spikes_v)

    zeros16 = jnp.zeros((_LANES,), jnp.float32)
    ones16 = jnp.ones((_LANES,), jnp.float32)

    def zero_row(t, carry):
        for g in range(_GROUPS):
            buf[t, pl.ds(g * _LANES, _LANES)] = zeros16
        return carry

    lax.fori_loop(0, _SIZE, zero_row, 0)

    def scatter_chunk(off, val):
        for g in range(_GROUPS):
            sp = spikes_v[pl.ds(off + g * _LANES, _LANES)]
            bins = (sp / jnp.float32(_TIME_STEP)).astype(jnp.int32)
            m = bins < _SIZE
            cols = lax.iota(jnp.int32, _LANES) + g * _LANES
            plsc.store_scatter(buf, [bins, cols], val, mask=m)

    def chunk_fn(k, carry):
        off = k * _CHUNK
        scatter_chunk(off, ones16)
        pltpu.sync_copy(
            buf, out_hbm.at[ch, :, pl.ds(s0 + off, _CHUNK)]
        )
        scatter_chunk(off, zeros16)
        return carry

    lax.fori_loop(0, _N_CHUNKS, chunk_fn, 0)


def kernel(spikes):
    flat = spikes.reshape(-1)
    run = pl.kernel(
        _sc_body,
        out_type=jax.ShapeDtypeStruct((_C, _SIZE, _SPATIAL), jnp.float32),
        mesh=plsc.VectorSubcoreMesh(core_axis_name="c", subcore_axis_name="s"),
        compiler_params=pltpu.CompilerParams(
            use_tc_tiling_on_sc=False, needs_layout_passes=False
        ),
        scratch_types=[
            pltpu.VMEM((_S_PER_W,), jnp.float32),
            pltpu.VMEM((_SIZE, _CHUNK), jnp.float32),
        ],
    )
    out = run(flat)
    return out.reshape(_C, _SIZE, _H, _W)


# speedup vs baseline: 1.0185x; 1.0185x over previous
import jax
import jax.numpy as jnp
from jax import lax
from jax.experimental import pallas as pl
from jax.experimental.pallas import tpu as pltpu
from jax.experimental.pallas import tpu_sc as plsc

_TIME_STEP = 0.002
_SIZE = 100
_NC, _NS = 2, 16
_NW = _NC * _NS
_C, _H, _W = 4, 256, 256
_SPATIAL = _H * _W
_S_PER_W = _C * _SPATIAL // _NW
_CHUNK = 512
_N_CHUNKS = _S_PER_W // _CHUNK
_LANES = 16
_GROUPS = _CHUNK // _LANES


def _sc_body(spikes_hbm, out_hbm, spikes_v, buf0, buf1, sem0, sem1):
    wid = lax.axis_index("s") * _NC + lax.axis_index("c")
    base = wid * _S_PER_W
    ch = base // _SPATIAL
    s0 = base % _SPATIAL
    bufs = (buf0, buf1)
    sems = (sem0, sem1)

    pltpu.sync_copy(spikes_hbm.at[pl.ds(base, _S_PER_W)], spikes_v)

    zeros16 = jnp.zeros((_LANES,), jnp.float32)
    ones16 = jnp.ones((_LANES,), jnp.float32)

    def zero_row(t, carry):
        for g in range(_GROUPS):
            buf0[t, pl.ds(g * _LANES, _LANES)] = zeros16
            buf1[t, pl.ds(g * _LANES, _LANES)] = zeros16
        return carry

    lax.fori_loop(0, _SIZE, zero_row, 0)

    def scatter_chunk(buf, off, val):
        for g in range(_GROUPS):
            sp = spikes_v[pl.ds(off + g * _LANES, _LANES)]
            bins = (sp / jnp.float32(_TIME_STEP)).astype(jnp.int32)
            m = bins < _SIZE
            cols = lax.iota(jnp.int32, _LANES) + g * _LANES
            plsc.store_scatter(buf, [bins, cols], val, mask=m)

    def dma(b, off):
        return pltpu.make_async_copy(
            bufs[b], out_hbm.at[ch, :, pl.ds(s0 + off, _CHUNK)], sems[b]
        )

    for b in range(2):
        scatter_chunk(bufs[b], b * _CHUNK, ones16)
        dma(b, b * _CHUNK).start()

    def loop_body(k2, carry):
        for b in range(2):
            k = 2 * k2 + b
            off = k * _CHUNK
            off_prev = off - 2 * _CHUNK
            dma(b, off_prev).wait()
            scatter_chunk(bufs[b], off_prev, zeros16)
            scatter_chunk(bufs[b], off, ones16)
            dma(b, off).start()
        return carry

    lax.fori_loop(1, _N_CHUNKS // 2, loop_body, 0)

    for b in range(2):
        dma(b, (_N_CHUNKS - 2 + b) * _CHUNK).wait()


def kernel(spikes):
    flat = spikes.reshape(-1)
    run = pl.kernel(
        _sc_body,
        out_type=jax.ShapeDtypeStruct((_C, _SIZE, _SPATIAL), jnp.float32),
        mesh=plsc.VectorSubcoreMesh(core_axis_name="c", subcore_axis_name="s"),
        compiler_params=pltpu.CompilerParams(
            use_tc_tiling_on_sc=False, needs_layout_passes=False
        ),
        scratch_types=[
            pltpu.VMEM((_S_PER_W,), jnp.float32),
            pltpu.VMEM((_SIZE, _CHUNK), jnp.float32),
            pltpu.VMEM((_SIZE, _CHUNK), jnp.float32),
            pltpu.SemaphoreType.DMA,
            pltpu.SemaphoreType.DMA,
        ],
    )
    out = run(flat)
    return out.reshape(_C, _SIZE, _H, _W)
